# baseline (speedup 1.0000x reference)
import functools

import jax
import jax.numpy as jnp
from jax import lax
from jax.experimental import pallas as pl
from jax.experimental.pallas import tpu as pltpu
from jax.experimental.pallas import tpu_sc as plsc


def kernel(input_tensor, weight):
    B0, B1 = input_tensor.shape
    V, D = weight.shape
    B = B0 * B1

    info = plsc.get_sparse_core_info()
    NC, NS = info.num_cores, info.num_subcores
    NW = NC * NS

    KW = 128
    CH = 8
    ROWS = CH * KW
    assert B % (NW * ROWS) == 0
    per_w = B // NW
    n_outer = per_w // ROWS

    idx = input_tensor.reshape(B // KW, KW).astype(jnp.int32)
    mesh = plsc.VectorSubcoreMesh(core_axis_name="c", subcore_axis_name="s")

    @functools.partial(
        pl.kernel,
        mesh=mesh,
        out_type=jax.ShapeDtypeStruct((B, D), jnp.float32),
        scratch_types=[
            pltpu.VMEM((CH, KW), jnp.int32),
            pltpu.VMEM((ROWS, D), jnp.float32),
            pltpu.SemaphoreType.DMA,
        ],
        compiler_params=pltpu.CompilerParams(use_tc_tiling_on_sc=False),
    )
    def emb(idx_hbm, table_hbm, out_hbm, idx_v, rows_v, sem):
        wid = lax.axis_index("s") * NC + lax.axis_index("c")
        idx_row0 = wid * (per_w // KW)
        out_row0 = wid * per_w

        def body(g, carry):
            pltpu.sync_copy(idx_hbm.at[pl.ds(idx_row0 + g * CH, CH), :], idx_v)
            for j in range(CH):
                pltpu.async_copy(
                    table_hbm.at[idx_v.at[j]],
                    rows_v.at[pl.ds(j * KW, KW), :],
                    sem,
                ).wait()
            pltpu.sync_copy(
                rows_v, out_hbm.at[pl.ds(out_row0 + g * ROWS, ROWS), :]
            )
            return carry

        lax.fori_loop(0, n_outer, body, 0)

    out = emb(idx, weight)
    return out.reshape(B0, B1, D)


# speedup vs baseline: 1.1171x; 1.1171x over previous
import functools

import jax
import jax.numpy as jnp
from jax import lax
from jax.experimental import pallas as pl
from jax.experimental.pallas import tpu as pltpu
from jax.experimental.pallas import tpu_sc as plsc


def kernel(input_tensor, weight):
    B0, B1 = input_tensor.shape
    V, D = weight.shape
    B = B0 * B1

    info = plsc.get_sparse_core_info()
    NC, NS = info.num_cores, info.num_subcores
    NW = NC * NS

    KW = 128
    CH = 4
    ROWS = CH * KW
    assert B % (NW * 2 * ROWS) == 0
    per_w = B // NW
    n_chunks = per_w // ROWS
    H = n_chunks // 2
    idx_rows_w = per_w // KW

    idx = input_tensor.reshape(B // KW, KW).astype(jnp.int32)
    mesh = plsc.VectorSubcoreMesh(core_axis_name="c", subcore_axis_name="s")

    @functools.partial(
        pl.kernel,
        mesh=mesh,
        out_type=jax.ShapeDtypeStruct((B, D), jnp.float32),
        scratch_types=[
            pltpu.VMEM((idx_rows_w, KW), jnp.int32),
            pltpu.VMEM((ROWS, D), jnp.float32),
            pltpu.VMEM((ROWS, D), jnp.float32),
            pltpu.SemaphoreType.DMA,
            pltpu.SemaphoreType.DMA,
            pltpu.SemaphoreType.DMA,
            pltpu.SemaphoreType.DMA,
        ],
        compiler_params=pltpu.CompilerParams(use_tc_tiling_on_sc=False),
    )
    def emb(idx_hbm, table_hbm, out_hbm, idx_v, rows0, rows1,
            g0, g1, o0, o1):
        wid = lax.axis_index("s") * NC + lax.axis_index("c")
        out_row0 = wid * per_w

        pltpu.sync_copy(idx_hbm.at[pl.ds(wid * idx_rows_w, idx_rows_w), :],
                        idx_v)

        def fire_gathers(chunk, rows_v, sem):
            for i in range(CH):
                pltpu.make_async_copy(
                    table_hbm.at[idx_v.at[chunk * CH + i]],
                    rows_v.at[pl.ds(i * KW, KW), :],
                    sem,
                ).start()

        def drain_gathers(rows_v, sem):
            pltpu.make_async_copy(
                out_hbm.at[pl.ds(0, ROWS), :], rows_v, sem).wait()

        def fire_write(chunk, rows_v, sem):
            pltpu.make_async_copy(
                rows_v,
                out_hbm.at[pl.ds(out_row0 + chunk * ROWS, ROWS), :],
                sem,
            ).start()

        def drain_write(rows_v, sem):
            pltpu.make_async_copy(
                rows_v, out_hbm.at[pl.ds(0, ROWS), :], sem).wait()

        fire_gathers(0, rows0, g0)
        fire_gathers(1, rows1, g1)

        def body(j, carry):
            drain_gathers(rows0, g0)
            fire_write(2 * j - 2, rows0, o0)
            drain_write(rows0, o0)
            fire_gathers(2 * j, rows0, g0)
            drain_gathers(rows1, g1)
            fire_write(2 * j - 1, rows1, o1)
            drain_write(rows1, o1)
            fire_gathers(2 * j + 1, rows1, g1)
            return carry

        lax.fori_loop(1, H, body, 0)

        drain_gathers(rows0, g0)
        fire_write(2 * H - 2, rows0, o0)
        drain_gathers(rows1, g1)
        fire_write(2 * H - 1, rows1, o1)
        drain_write(rows0, o0)
        drain_write(rows1, o1)

    out = emb(idx, weight)
    return out.reshape(B0, B1, D)
